# Initial kernel scaffold; baseline (speedup 1.0000x reference)
import jax
import jax.numpy as jnp
from jax.experimental import pallas as pl


def kernel(feature_input, inputs, Wexp, bexp, bn_gamma, bn_beta, extra_bias, Wg, bg, gbias, global_weights):
    raise NotImplementedError("write your pallas kernel here")


# speedup vs baseline: 8.7245x; 8.7245x over previous
import functools

import jax
import jax.numpy as jnp
from jax.experimental import pallas as pl

UNITS = 64
E = 16
D_EXP = 128
FEAT = 1024
H = 4
TOPK = 4
BN_EPS = 1e-5
TB = 512


def _moe_block(feat_ref, x_ref, wexp_ref, bexp_ref, gamma_ref, beta_ref,
               ebias_ref, wgm_ref, gb_ref, gw_ref, out_ref, eo_ref):
    inv_bn = 1.0 / jnp.sqrt(1.0 + BN_EPS)

    for e in range(E):
        x_e = x_ref[:, e * D_EXP:(e + 1) * D_EXP]
        w_e = wexp_ref[e]
        o = jax.lax.dot_general(
            x_e, w_e, (((1,), (1,)), ((), ())),
            preferred_element_type=jnp.float32)
        o = o + bexp_ref[e:e + 1, :]
        o = o * (inv_bn * gamma_ref[e:e + 1, :]) + beta_ref[e:e + 1, :]
        o = jnp.maximum(o, 0.0)
        o = o + ebias_ref[e:e + 1, :]
        o = jnp.where(o >= 0.0, o, 0.01 * o)
        eo_ref[:, e * UNITS:(e + 1) * UNITS] = o

    g = jax.lax.dot_general(
        feat_ref[...], wgm_ref[...], (((1,), (1,)), ((), ())),
        preferred_element_type=jnp.float32)
    g = jnp.maximum(g + gb_ref[...], 0.0)

    gw = gw_ref[...]
    m = jnp.max(gw, axis=-1, keepdims=True)
    egw = jnp.exp((gw - m) / 0.01)
    ngw = egw / jnp.sum(egw, axis=-1, keepdims=True)

    iota = jax.lax.broadcasted_iota(jnp.int32, (g.shape[0], E), 1)
    for i in range(H):
        wg = g[:, i * E:(i + 1) * E] * ngw
        vals = wg
        topv, onehots = [], []
        for _ in range(TOPK):
            mk = jnp.max(vals, axis=-1, keepdims=True)
            idx = jnp.min(jnp.where(vals == mk, iota, E),
                          axis=-1, keepdims=True)
            sel = iota == idx
            topv.append(mk)
            onehots.append(sel)
            vals = jnp.where(sel, -jnp.inf, vals)
        exps = [jnp.exp((v - topv[0]) / 0.01) for v in topv]
        denom = exps[0] + exps[1] + exps[2] + exps[3]
        wdense = jnp.zeros_like(wg)
        for k in range(TOPK):
            wdense = wdense + jnp.where(onehots[k], exps[k] / denom, 0.0)
        head = jnp.zeros((g.shape[0], UNITS), dtype=jnp.float32)
        for e in range(E):
            head = head + wdense[:, e:e + 1] * eo_ref[:, e * UNITS:(e + 1) * UNITS]
        out_ref[:, i * UNITS:(i + 1) * UNITS] = head


@functools.partial(jax.jit, static_argnums=())
def kernel(feature_input, inputs, Wexp, bexp, bn_gamma, bn_beta, extra_bias,
           Wg, bg, gbias, global_weights):
    B = feature_input.shape[0]
    wgm = Wg.reshape(H * E, FEAT)
    gb = (bg + gbias).reshape(1, H * E)
    gw = global_weights.reshape(1, E)

    grid = (B // TB,)
    out, eo = pl.pallas_call(
        _moe_block,
        grid=grid,
        in_specs=[
            pl.BlockSpec((TB, FEAT), lambda i: (i, 0)),
            pl.BlockSpec((TB, E * D_EXP), lambda i: (i, 0)),
            pl.BlockSpec((E, UNITS, D_EXP), lambda i: (0, 0, 0)),
            pl.BlockSpec((E, UNITS), lambda i: (0, 0)),
            pl.BlockSpec((E, UNITS), lambda i: (0, 0)),
            pl.BlockSpec((E, UNITS), lambda i: (0, 0)),
            pl.BlockSpec((E, UNITS), lambda i: (0, 0)),
            pl.BlockSpec((H * E, FEAT), lambda i: (0, 0)),
            pl.BlockSpec((1, H * E), lambda i: (0, 0)),
            pl.BlockSpec((1, E), lambda i: (0, 0)),
        ],
        out_specs=[
            pl.BlockSpec((TB, H * UNITS), lambda i: (i, 0)),
            pl.BlockSpec((TB, E * UNITS), lambda i: (i, 0)),
        ],
        out_shape=[
            jax.ShapeDtypeStruct((B, H * UNITS), jnp.float32),
            jax.ShapeDtypeStruct((B, E * UNITS), jnp.float32),
        ],
    )(feature_input, inputs, Wexp, bexp, bn_gamma, bn_beta, extra_bias,
      wgm, gb, gw)
    return (out, eo.reshape(B, E, UNITS).transpose(0, 2, 1))


# speedup vs baseline: 25.8461x; 2.9625x over previous
import jax
import jax.numpy as jnp
from jax.experimental import pallas as pl

UNITS = 64
E = 16
D_EXP = 128
FEAT = 1024
H = 4
TOPK = 4
BN_EPS = 1e-5
TB = 512


def _moe_block(feat_ref, x_ref, wexp_ref, bexpT_ref, gammaT_ref, betaT_ref,
               ebiasT_ref, wgm_ref, gbT_ref, gwT_ref, outT_ref, eoT_ref):
    inv_bn = 1.0 / jnp.sqrt(1.0 + BN_EPS)
    nt = feat_ref.shape[0]

    gT = jax.lax.dot_general(
        wgm_ref[...], feat_ref[...], (((1,), (1,)), ((), ())),
        preferred_element_type=jnp.float32)
    gT = jnp.maximum(gT + gbT_ref[...], 0.0)

    gwT = gwT_ref[...]
    m = jnp.max(gwT, axis=0, keepdims=True)
    egw = jnp.exp((gwT - m) / 0.01)
    ngwT = egw / jnp.sum(egw, axis=0, keepdims=True)

    siota = jax.lax.broadcasted_iota(jnp.int32, (E, nt), 0)
    wdense = []
    for i in range(H):
        vals = gT[i * E:(i + 1) * E, :] * ngwT
        topv, onehots = [], []
        for _ in range(TOPK):
            mk = jnp.max(vals, axis=0, keepdims=True)
            idx = jnp.min(jnp.where(vals == mk, siota, E),
                          axis=0, keepdims=True)
            sel = siota == idx
            topv.append(mk)
            onehots.append(sel)
            vals = jnp.where(sel, -1e30, vals)
        exps = [jnp.exp((v - topv[0]) / 0.01) for v in topv]
        denom = exps[0] + exps[1] + exps[2] + exps[3]
        wd = jnp.zeros((E, nt), dtype=jnp.float32)
        for k in range(TOPK):
            wd = wd + jnp.where(onehots[k], exps[k] / denom, 0.0)
        wdense.append(wd)

    heads = [jnp.zeros((UNITS, nt), dtype=jnp.float32) for _ in range(H)]
    for e in range(E):
        x_e = x_ref[:, e * D_EXP:(e + 1) * D_EXP]
        oT = jax.lax.dot_general(
            wexp_ref[e], x_e, (((1,), (1,)), ((), ())),
            preferred_element_type=jnp.float32)
        oT = oT + bexpT_ref[:, e:e + 1]
        oT = oT * (inv_bn * gammaT_ref[:, e:e + 1]) + betaT_ref[:, e:e + 1]
        oT = jnp.maximum(oT, 0.0)
        oT = oT + ebiasT_ref[:, e:e + 1]
        oT = jnp.where(oT >= 0.0, oT, 0.01 * oT)
        eoT_ref[e * UNITS:(e + 1) * UNITS, :] = oT
        for i in range(H):
            heads[i] = heads[i] + wdense[i][e:e + 1, :] * oT
    for i in range(H):
        outT_ref[i * UNITS:(i + 1) * UNITS, :] = heads[i]


def kernel(feature_input, inputs, Wexp, bexp, bn_gamma, bn_beta, extra_bias,
           Wg, bg, gbias, global_weights):
    B = feature_input.shape[0]
    wgm = Wg.reshape(H * E, FEAT)
    gbT = (bg + gbias).reshape(H * E, 1)
    gwT = global_weights.reshape(E, 1)
    bexpT = bexp.T
    gammaT = bn_gamma.T
    betaT = bn_beta.T
    ebiasT = extra_bias.T

    grid = (B // TB,)
    outT, eoT = pl.pallas_call(
        _moe_block,
        grid=grid,
        in_specs=[
            pl.BlockSpec((TB, FEAT), lambda i: (i, 0)),
            pl.BlockSpec((TB, E * D_EXP), lambda i: (i, 0)),
            pl.BlockSpec((E, UNITS, D_EXP), lambda i: (0, 0, 0)),
            pl.BlockSpec((UNITS, E), lambda i: (0, 0)),
            pl.BlockSpec((UNITS, E), lambda i: (0, 0)),
            pl.BlockSpec((UNITS, E), lambda i: (0, 0)),
            pl.BlockSpec((UNITS, E), lambda i: (0, 0)),
            pl.BlockSpec((H * E, FEAT), lambda i: (0, 0)),
            pl.BlockSpec((H * E, 1), lambda i: (0, 0)),
            pl.BlockSpec((E, 1), lambda i: (0, 0)),
        ],
        out_specs=[
            pl.BlockSpec((H * UNITS, TB), lambda i: (0, i)),
            pl.BlockSpec((E * UNITS, TB), lambda i: (0, i)),
        ],
        out_shape=[
            jax.ShapeDtypeStruct((H * UNITS, B), jnp.float32),
            jax.ShapeDtypeStruct((E * UNITS, B), jnp.float32),
        ],
    )(feature_input, inputs, Wexp, bexpT, gammaT, betaT, ebiasT,
      wgm, gbT, gwT)
    out = outT.T
    eo = eoT.reshape(E, UNITS, B).transpose(2, 1, 0)
    return (out, eo)


# speedup vs baseline: 26.3881x; 1.0210x over previous
import jax
import jax.numpy as jnp
from jax.experimental import pallas as pl

UNITS = 64
E = 16
D_EXP = 128
FEAT = 1024
H = 4
TOPK = 4
BN_EPS = 1e-5
TB = 1024


def _moe_block(feat_ref, x_ref, wexp_ref, bexpT_ref, gammaT_ref, betaT_ref,
               ebiasT_ref, wgm_ref, gbT_ref, gwT_ref, outT_ref, eoT_ref):
    inv_bn = 1.0 / jnp.sqrt(1.0 + BN_EPS)
    nt = feat_ref.shape[0]

    gT = jax.lax.dot_general(
        wgm_ref[...], feat_ref[...], (((1,), (1,)), ((), ())),
        preferred_element_type=jnp.float32)
    gT = jnp.maximum(gT + gbT_ref[...], 0.0)

    gwT = gwT_ref[...]
    m = jnp.max(gwT, axis=0, keepdims=True)
    egw = jnp.exp((gwT - m) / 0.01)
    ngwT = egw / jnp.sum(egw, axis=0, keepdims=True)

    siota = jax.lax.broadcasted_iota(jnp.int32, (E, nt), 0)
    wdense = []
    for i in range(H):
        vals = gT[i * E:(i + 1) * E, :] * ngwT
        topv, onehots = [], []
        for _ in range(TOPK):
            mk = jnp.max(vals, axis=0, keepdims=True)
            idx = jnp.min(jnp.where(vals == mk, siota, E),
                          axis=0, keepdims=True)
            sel = siota == idx
            topv.append(mk)
            onehots.append(sel)
            vals = jnp.where(sel, -1e30, vals)
        exps = [jnp.exp((v - topv[0]) / 0.01) for v in topv]
        denom = exps[0] + exps[1] + exps[2] + exps[3]
        wd = jnp.zeros((E, nt), dtype=jnp.float32)
        for k in range(TOPK):
            wd = wd + jnp.where(onehots[k], exps[k] / denom, 0.0)
        wdense.append(wd)

    heads = [jnp.zeros((UNITS, nt), dtype=jnp.float32) for _ in range(H)]
    for e in range(E):
        x_e = x_ref[:, e * D_EXP:(e + 1) * D_EXP]
        oT = jax.lax.dot_general(
            wexp_ref[e], x_e, (((1,), (1,)), ((), ())),
            preferred_element_type=jnp.float32)
        oT = oT + bexpT_ref[:, e:e + 1]
        oT = oT * (inv_bn * gammaT_ref[:, e:e + 1]) + betaT_ref[:, e:e + 1]
        oT = jnp.maximum(oT, 0.0)
        oT = oT + ebiasT_ref[:, e:e + 1]
        oT = jnp.where(oT >= 0.0, oT, 0.01 * oT)
        eoT_ref[e * UNITS:(e + 1) * UNITS, :] = oT
        for i in range(H):
            heads[i] = heads[i] + wdense[i][e:e + 1, :] * oT
    for i in range(H):
        outT_ref[i * UNITS:(i + 1) * UNITS, :] = heads[i]


def kernel(feature_input, inputs, Wexp, bexp, bn_gamma, bn_beta, extra_bias,
           Wg, bg, gbias, global_weights):
    B = feature_input.shape[0]
    wgm = Wg.reshape(H * E, FEAT)
    gbT = (bg + gbias).reshape(H * E, 1)
    gwT = global_weights.reshape(E, 1)
    bexpT = bexp.T
    gammaT = bn_gamma.T
    betaT = bn_beta.T
    ebiasT = extra_bias.T

    grid = (B // TB,)
    outT, eoT = pl.pallas_call(
        _moe_block,
        grid=grid,
        in_specs=[
            pl.BlockSpec((TB, FEAT), lambda i: (i, 0)),
            pl.BlockSpec((TB, E * D_EXP), lambda i: (i, 0)),
            pl.BlockSpec((E, UNITS, D_EXP), lambda i: (0, 0, 0)),
            pl.BlockSpec((UNITS, E), lambda i: (0, 0)),
            pl.BlockSpec((UNITS, E), lambda i: (0, 0)),
            pl.BlockSpec((UNITS, E), lambda i: (0, 0)),
            pl.BlockSpec((UNITS, E), lambda i: (0, 0)),
            pl.BlockSpec((H * E, FEAT), lambda i: (0, 0)),
            pl.BlockSpec((H * E, 1), lambda i: (0, 0)),
            pl.BlockSpec((E, 1), lambda i: (0, 0)),
        ],
        out_specs=[
            pl.BlockSpec((H * UNITS, TB), lambda i: (0, i)),
            pl.BlockSpec((E * UNITS, TB), lambda i: (0, i)),
        ],
        out_shape=[
            jax.ShapeDtypeStruct((H * UNITS, B), jnp.float32),
            jax.ShapeDtypeStruct((E * UNITS, B), jnp.float32),
        ],
    )(feature_input, inputs, Wexp, bexpT, gammaT, betaT, ebiasT,
      wgm, gbT, gwT)
    out = outT.T
    eo = eoT.reshape(E, UNITS, B).transpose(2, 1, 0)
    return (out, eo)
